# baseline (speedup 1.0000x reference)
import functools

import jax
import jax.numpy as jnp
from jax import lax
from jax.experimental import pallas as pl
from jax.experimental.pallas import tpu as pltpu
from jax.experimental.pallas import tpu_sc as plsc

HIDDEN = 4096
E = 64
TOPK = 8
N_GROUPS = 8
EPG = E // N_GROUPS
TOPK_GROUPS = 4
SCALE = 2.5

NW = 32
L = 16


def _matmul_kernel(w_ref, x_ref, s_ref):
    s_ref[...] = jax.nn.sigmoid(
        jax.lax.dot_general(w_ref[...], x_ref[...], (((1,), (1,)), ((), ())),
                            preferred_element_type=jnp.float32))


def _scores_ET(x_TD, kernel_DE):
    T = x_TD.shape[0]
    TT = 1024
    wt = kernel_DE.T
    return pl.pallas_call(
        _matmul_kernel,
        grid=(T // TT,),
        in_specs=[
            pl.BlockSpec((E, HIDDEN), lambda i: (0, 0)),
            pl.BlockSpec((TT, HIDDEN), lambda i: (i, 0)),
        ],
        out_specs=pl.BlockSpec((E, TT), lambda i: (0, i)),
        out_shape=jax.ShapeDtypeStruct((E, T), jnp.float32),
    )(wt, x_TD)


def _sc_select_kernel(scores_hbm, bias_hbm, wout_hbm, iout_hbm,
                      chunk_v, bias_v, wbuf_v, ibuf_v):
    T_PER_W = scores_hbm.shape[1] // NW
    NB = T_PER_W // L
    wid = lax.axis_index("s") * 2 + lax.axis_index("c")
    base = wid * T_PER_W
    pltpu.sync_copy(scores_hbm.at[:, pl.ds(base, T_PER_W)], chunk_v)
    pltpu.sync_copy(bias_hbm, bias_v)
    neg = jnp.float32(-jnp.inf)

    def body(b, carry):
        tok = b * L

        gs_list = []
        for g in range(N_GROUPS):
            vs = []
            for j in range(EPG):
                e = g * EPG + j
                vs.append(chunk_v[e, pl.ds(tok, L)] + bias_v[e, :])
            m1 = vs[0]
            for j in range(1, EPG):
                m1 = jnp.maximum(m1, vs[j])
            cnt = jnp.zeros((L,), jnp.float32)
            m2 = jnp.full((L,), neg)
            for j in range(EPG):
                eq = vs[j] == m1
                cnt = cnt + jnp.where(eq, 1.0, 0.0)
                m2 = jnp.maximum(m2, jnp.where(eq, neg, vs[j]))
            m2 = jnp.where(cnt >= 2.0, m1, m2)
            gs_list.append(m1 + m2)

        keep = [jnp.zeros((L,), jnp.bool_) for _ in range(N_GROUPS)]
        gs_w = list(gs_list)
        for _ in range(TOPK_GROUPS):
            m = gs_w[0]
            for g in range(1, N_GROUPS):
                m = jnp.maximum(m, gs_w[g])
            gidx = jnp.full((L,), N_GROUPS, jnp.int32)
            for g in range(N_GROUPS - 1, -1, -1):
                gidx = jnp.where(gs_w[g] == m, jnp.full((L,), g, jnp.int32),
                                 gidx)
            for g in range(N_GROUPS):
                sel = gidx == g
                keep[g] = jnp.logical_or(keep[g], sel)
                gs_w[g] = jnp.where(sel, neg, gs_w[g])

        ms = [jnp.full((L,), neg) for _ in range(TOPK)]
        ws = [jnp.zeros((L,), jnp.float32) for _ in range(TOPK)]
        es = [jnp.zeros((L,), jnp.int32) for _ in range(TOPK)]
        for e in range(E):
            raw = chunk_v[e, pl.ds(tok, L)]
            v = raw + bias_v[e, :]
            v = jnp.where(keep[e // EPG], v, 0.0)
            e_spl = jnp.full((L,), e, jnp.int32)
            cs = [v > ms[r] for r in range(TOPK)]
            for r in range(TOPK - 1, 0, -1):
                ms[r] = jnp.where(cs[r - 1], ms[r - 1],
                                  jnp.where(cs[r], v, ms[r]))
                ws[r] = jnp.where(cs[r - 1], ws[r - 1],
                                  jnp.where(cs[r], raw, ws[r]))
                es[r] = jnp.where(cs[r - 1], es[r - 1],
                                  jnp.where(cs[r], e_spl, es[r]))
            ms[0] = jnp.where(cs[0], v, ms[0])
            ws[0] = jnp.where(cs[0], raw, ws[0])
            es[0] = jnp.where(cs[0], e_spl, es[0])

        wsum = ws[0]
        for k in range(1, TOPK):
            wsum = wsum + ws[k]
        wsum = wsum + 1e-20
        for k in range(TOPK):
            wbuf_v[k, pl.ds(tok, L)] = ws[k] / wsum * SCALE
            ibuf_v[k, pl.ds(tok, L)] = es[k]
        return carry

    lax.fori_loop(0, NB, body, None)
    pltpu.sync_copy(wbuf_v, wout_hbm.at[:, pl.ds(base, T_PER_W)])
    pltpu.sync_copy(ibuf_v, iout_hbm.at[:, pl.ds(base, T_PER_W)])


def _sc_select(scores_ET, bias16):
    T = scores_ET.shape[1]
    tpw = T // NW
    mesh = plsc.VectorSubcoreMesh(core_axis_name="c", subcore_axis_name="s")
    k = functools.partial(
        pl.kernel,
        mesh=mesh,
        out_type=[
            jax.ShapeDtypeStruct((TOPK, T), jnp.float32),
            jax.ShapeDtypeStruct((TOPK, T), jnp.int32),
        ],
        scratch_types=[
            pltpu.VMEM((E, tpw), jnp.float32),
            pltpu.VMEM((E, L), jnp.float32),
            pltpu.VMEM((TOPK, tpw), jnp.float32),
            pltpu.VMEM((TOPK, tpw), jnp.int32),
        ],
    )(_sc_select_kernel)
    return k(scores_ET, bias16)


@jax.jit
def kernel(x_TD, kernel_DE, bias_E):
    x_TD = jnp.asarray(x_TD, jnp.float32)
    T = x_TD.shape[0]
    scores = _scores_ET(x_TD, kernel_DE)
    bias16 = jnp.broadcast_to(bias_E.astype(jnp.float32)[:, None], (E, L))
    w_kT, i_kT = _sc_select(scores, bias16)
    return w_kT.T, i_kT.T


# speedup vs baseline: 1.3088x; 1.3088x over previous
import jax
import jax.numpy as jnp
from jax.experimental import pallas as pl

HIDDEN = 4096
E = 64
TOPK = 8
N_GROUPS = 8
EPG = E // N_GROUPS
TOPK_GROUPS = 4
SCALE = 2.5


def _router_kernel(wt_ref, b_ref, x_ref, wout_ref, iout_ref):
    x = x_ref[...]
    wt = wt_ref[...]
    TT = x.shape[0]
    scores = jax.nn.sigmoid(
        jax.lax.dot_general(wt, x, (((1,), (1,)), ((), ())),
                            preferred_element_type=jnp.float32))
    s = scores + b_ref[...]
    neg = jnp.float32(-jnp.inf)
    iota_e = jax.lax.broadcasted_iota(jnp.int32, (E, TT), 0)
    grp_of_e = iota_e // EPG
    iota_g = jax.lax.broadcasted_iota(jnp.int32, (N_GROUPS, TT), 0)

    gs_rows = []
    for g in range(N_GROUPS):
        sg = s[g * EPG:(g + 1) * EPG, :]
        m1 = jnp.max(sg, axis=0, keepdims=True)
        ismax = sg == m1
        cnt = jnp.sum(ismax.astype(jnp.float32), axis=0, keepdims=True)
        m2 = jnp.max(jnp.where(ismax, neg, sg), axis=0, keepdims=True)
        m2 = jnp.where(cnt >= 2.0, m1, m2)
        gs_rows.append(m1 + m2)
    gs = jnp.concatenate(gs_rows, axis=0)

    mask_e = jnp.zeros((E, TT), dtype=jnp.bool_)
    for _ in range(TOPK_GROUPS):
        m = jnp.max(gs, axis=0, keepdims=True)
        gidx = jnp.min(jnp.where(gs == m, iota_g, N_GROUPS), axis=0,
                       keepdims=True)
        mask_e = jnp.logical_or(mask_e, grp_of_e == gidx)
        gs = jnp.where(iota_g == gidx, neg, gs)

    sm = jnp.where(mask_e, s, 0.0)
    wrows = []
    irows = []
    for _ in range(TOPK):
        m = jnp.max(sm, axis=0, keepdims=True)
        eidx = jnp.min(jnp.where(sm == m, iota_e, E), axis=0, keepdims=True)
        sel = iota_e == eidx
        wrows.append(jnp.max(jnp.where(sel, scores, neg), axis=0,
                             keepdims=True))
        irows.append(eidx)
        sm = jnp.where(sel, neg, sm)
    wts = jnp.concatenate(wrows, axis=0)
    idxs = jnp.concatenate(irows, axis=0)
    wts = wts / (jnp.sum(wts, axis=0, keepdims=True) + 1e-20) * SCALE
    wout_ref[...] = wts.T
    iout_ref[...] = idxs.T


@jax.jit
def kernel(x_TD, kernel_DE, bias_E):
    x_TD = jnp.asarray(x_TD, jnp.float32)
    T = x_TD.shape[0]
    TT = 1024
    wt = kernel_DE.T
    b = bias_E.reshape(E, 1).astype(jnp.float32)
    return pl.pallas_call(
        _router_kernel,
        grid=(T // TT,),
        in_specs=[
            pl.BlockSpec((E, HIDDEN), lambda i: (0, 0)),
            pl.BlockSpec((E, 1), lambda i: (0, 0)),
            pl.BlockSpec((TT, HIDDEN), lambda i: (i, 0)),
        ],
        out_specs=[
            pl.BlockSpec((TT, TOPK), lambda i: (i, 0)),
            pl.BlockSpec((TT, TOPK), lambda i: (i, 0)),
        ],
        out_shape=[
            jax.ShapeDtypeStruct((T, TOPK), jnp.float32),
            jax.ShapeDtypeStruct((T, TOPK), jnp.int32),
        ],
    )(wt, b, x_TD)
